# baseline (speedup 1.0000x reference)
import functools

import jax
import jax.numpy as jnp
from jax import lax
from jax.experimental import pallas as pl
from jax.experimental.pallas import tpu as pltpu
from jax.experimental.pallas import tpu_sc as plsc

NUM_CORES = 2
NUM_SUBCORES = 16
NW = NUM_CORES * NUM_SUBCORES

ROWS_PER_GATHER = 128
GPB = 4
EMB_DIM = 64


def _make_sc_gather(n_chunks: int):
  n_groups = n_chunks // GPB

  @functools.partial(
      pl.kernel,
      out_type=jax.ShapeDtypeStruct(
          (NW, n_chunks, ROWS_PER_GATHER, EMB_DIM), jnp.float32
      ),
      mesh=plsc.VectorSubcoreMesh(
          core_axis_name="c",
          subcore_axis_name="s",
          num_cores=NUM_CORES,
          num_subcores=NUM_SUBCORES,
      ),
      scratch_types=[
          pltpu.VMEM((n_chunks, ROWS_PER_GATHER), jnp.int32),
          pltpu.VMEM((GPB, ROWS_PER_GATHER, EMB_DIM), jnp.float32),
          pltpu.VMEM((GPB, ROWS_PER_GATHER, EMB_DIM), jnp.float32),
          pltpu.SemaphoreType.DMA,
          pltpu.SemaphoreType.DMA,
      ],
      compiler_params=pltpu.CompilerParams(use_tc_tiling_on_sc=False),
  )
  def gather_kernel(tok_hbm, w_hbm, out_hbm, idx_v, buf0, buf1, sem0, sem1):
    c = lax.axis_index("c")
    s = lax.axis_index("s")
    wid = s * NUM_CORES + c
    pltpu.sync_copy(tok_hbm.at[wid], idx_v)

    bufs = (buf0, buf1)
    sems = (sem0, sem1)

    def fire(g, b):
      for k in range(GPB):
        pltpu.async_copy(
            w_hbm.at[idx_v.at[g * GPB + k]], bufs[b].at[k], sems[b]
        )

    def drain_write(g, b):
      for k in range(GPB):
        pltpu.make_async_copy(
            w_hbm.at[idx_v.at[g * GPB + k]], bufs[b].at[k], sems[b]
        ).wait()
      pltpu.sync_copy(bufs[b], out_hbm.at[wid, pl.ds(g * GPB, GPB)])

    fire(0, 0)

    def step(t, carry):
      g = 2 * t
      fire(g + 1, 1)
      drain_write(g, 0)
      fire(g + 2, 0)
      drain_write(g + 1, 1)
      return carry

    lax.fori_loop(0, n_groups // 2 - 1, step, 0)
    g_last = n_groups - 2
    fire(g_last + 1, 1)
    drain_write(g_last, 0)
    drain_write(g_last + 1, 1)

  return gather_kernel


def kernel(token_ids, weight):
  n_tokens = token_ids.size
  assert n_tokens % (NW * ROWS_PER_GATHER) == 0
  n_chunks = n_tokens // (NW * ROWS_PER_GATHER)
  assert n_chunks % (2 * GPB) == 0

  tok = token_ids.astype(jnp.int32).reshape(NW, n_chunks, ROWS_PER_GATHER)
  out = _make_sc_gather(n_chunks)(tok, weight)
  return out.reshape(*token_ids.shape, EMB_DIM)


# speedup vs baseline: 1.0005x; 1.0005x over previous
import functools

import jax
import jax.numpy as jnp
from jax import lax
from jax.experimental import pallas as pl
from jax.experimental.pallas import tpu as pltpu
from jax.experimental.pallas import tpu_sc as plsc

NUM_CORES = 2
NUM_SUBCORES = 16
NW = NUM_CORES * NUM_SUBCORES

ROWS_PER_GATHER = 512
GPB = 1
EMB_DIM = 64


def _make_sc_gather(n_chunks: int):
  n_groups = n_chunks // GPB

  @functools.partial(
      pl.kernel,
      out_type=jax.ShapeDtypeStruct(
          (NW, n_chunks, ROWS_PER_GATHER, EMB_DIM), jnp.float32
      ),
      mesh=plsc.VectorSubcoreMesh(
          core_axis_name="c",
          subcore_axis_name="s",
          num_cores=NUM_CORES,
          num_subcores=NUM_SUBCORES,
      ),
      scratch_types=[
          pltpu.VMEM((n_chunks, ROWS_PER_GATHER), jnp.int32),
          pltpu.VMEM((GPB, ROWS_PER_GATHER, EMB_DIM), jnp.float32),
          pltpu.VMEM((GPB, ROWS_PER_GATHER, EMB_DIM), jnp.float32),
          pltpu.SemaphoreType.DMA,
          pltpu.SemaphoreType.DMA,
      ],
      compiler_params=pltpu.CompilerParams(use_tc_tiling_on_sc=False),
  )
  def gather_kernel(tok_hbm, w_hbm, out_hbm, idx_v, buf0, buf1, sem0, sem1):
    c = lax.axis_index("c")
    s = lax.axis_index("s")
    wid = s * NUM_CORES + c
    pltpu.sync_copy(tok_hbm.at[wid], idx_v)

    bufs = (buf0, buf1)
    sems = (sem0, sem1)

    def fire(g, b):
      for k in range(GPB):
        pltpu.async_copy(
            w_hbm.at[idx_v.at[g * GPB + k]], bufs[b].at[k], sems[b]
        )

    def drain_write(g, b):
      for k in range(GPB):
        pltpu.make_async_copy(
            w_hbm.at[idx_v.at[g * GPB + k]], bufs[b].at[k], sems[b]
        ).wait()
      pltpu.sync_copy(bufs[b], out_hbm.at[wid, pl.ds(g * GPB, GPB)])

    fire(0, 0)

    def step(t, carry):
      g = 2 * t
      fire(g + 1, 1)
      drain_write(g, 0)
      fire(g + 2, 0)
      drain_write(g + 1, 1)
      return carry

    lax.fori_loop(0, n_groups // 2 - 1, step, 0)
    g_last = n_groups - 2
    fire(g_last + 1, 1)
    drain_write(g_last, 0)
    drain_write(g_last + 1, 1)

  return gather_kernel


def kernel(token_ids, weight):
  n_tokens = token_ids.size
  assert n_tokens % (NW * ROWS_PER_GATHER) == 0
  n_chunks = n_tokens // (NW * ROWS_PER_GATHER)
  assert n_chunks % (2 * GPB) == 0

  tok = token_ids.astype(jnp.int32).reshape(NW, n_chunks, ROWS_PER_GATHER)
  out = _make_sc_gather(n_chunks)(tok, weight)
  return out.reshape(*token_ids.shape, EMB_DIM)
